# baseline (speedup 1.0000x reference)
import functools

import jax
import jax.numpy as jnp
from jax import lax
from jax.experimental import pallas as pl
from jax.experimental.pallas import tpu as pltpu
from jax.experimental.pallas import tpu_sc as plsc

_R, _C = 64, 32768
_NC, _NS, _L = 2, 16, 16
_NW = _NC * _NS
_GR = 32
_NG = _R // _GR
_PANEL = _C // (_NW // _NG)
_CHUNK = _PANEL // 2
_NCHUNK = 2

_mesh = plsc.VectorSubcoreMesh(core_axis_name="c", subcore_axis_name="s")


@functools.partial(
    pl.kernel,
    out_type=jax.ShapeDtypeStruct((_R, _C), jnp.float32),
    mesh=_mesh,
    scratch_types=[
        pltpu.VMEM((_NCHUNK, _GR, _CHUNK), jnp.float32),
        pltpu.VMEM((_NCHUNK, _CHUNK), jnp.int32),
        pltpu.SemaphoreType.DMA((_NCHUNK,)),
        pltpu.SemaphoreType.DMA((_NCHUNK,)),
        pltpu.SemaphoreType.DMA((_NCHUNK,)),
    ],
)
def _domain_mask_sc(w_hbm, p_hbm, out_hbm, w_v, b_v, s_w, s_b, s_o):
    wid = lax.axis_index("s") * _NC + lax.axis_index("c")
    g = wid % _NG
    r0 = pl.multiple_of(g * _GR, _GR)
    col0 = pl.multiple_of((wid // _NG) * _PANEL, _PANEL)
    pbase = pl.multiple_of(g * _C + col0, _CHUNK)

    zero = jnp.zeros((_L,), jnp.float32)

    in_w, in_b, out_h = [], [], []
    for c in range(_NCHUNK):
        in_w.append(pltpu.async_copy(
            w_hbm.at[pl.ds(r0, _GR), pl.ds(col0 + c * _CHUNK, _CHUNK)],
            w_v.at[c], s_w.at[c]))
        in_b.append(pltpu.async_copy(
            p_hbm.at[pl.ds(pbase + c * _CHUNK, _CHUNK)], b_v.at[c], s_b.at[c]))

    for c in range(_NCHUNK):
        in_w[c].wait()
        in_b[c].wait()

        @plsc.parallel_loop(0, _CHUNK // _L)
        def _body(j):
            words = b_v[c, pl.ds(j * _L, _L)]
            for r in range(_GR):
                vec = w_v[c, r, pl.ds(j * _L, _L)]
                hit = (words << (31 - r)) < 0
                w_v[c, r, pl.ds(j * _L, _L)] = jnp.where(hit, vec, zero)

        out_h.append(pltpu.async_copy(
            w_v.at[c],
            out_hbm.at[pl.ds(r0, _GR), pl.ds(col0 + c * _CHUNK, _CHUNK)],
            s_o.at[c]))

    for h in out_h:
        h.wait()


def _pack_rows(mask):
    m = mask.view(jnp.int8).reshape(_NG, _GR, _C).astype(jnp.uint32)
    wt = jnp.left_shift(
        jnp.uint32(1), jnp.arange(_GR, dtype=jnp.uint32)
    )[None, :, None]
    packed = (m * wt).sum(axis=1, dtype=jnp.uint32)
    return lax.bitcast_convert_type(packed, jnp.int32).reshape(_NG * _C)


def kernel(w, mask):
    return _domain_mask_sc(w, _pack_rows(mask))


# speedup vs baseline: 1.1821x; 1.1821x over previous
import functools

import jax
import jax.numpy as jnp
from jax import lax
from jax.experimental import pallas as pl
from jax.experimental.pallas import tpu as pltpu
from jax.experimental.pallas import tpu_sc as plsc

_R, _C = 64, 32768
_NC, _NS, _L = 2, 16, 16
_NW = _NC * _NS
_SC_R = 16
_PANEL = _C // _NW
_CHUNK = _PANEL // 2
_NCHUNK = 2

_mesh = plsc.VectorSubcoreMesh(core_axis_name="c", subcore_axis_name="s")


@functools.partial(
    pl.kernel,
    out_type=jax.ShapeDtypeStruct((_SC_R, _C), jnp.float32),
    mesh=_mesh,
    scratch_types=[
        pltpu.VMEM((_NCHUNK, _SC_R, _CHUNK), jnp.float32),
        pltpu.VMEM((_NCHUNK, _CHUNK), jnp.int32),
        pltpu.SemaphoreType.DMA((_NCHUNK,)),
        pltpu.SemaphoreType.DMA((_NCHUNK,)),
        pltpu.SemaphoreType.DMA((_NCHUNK,)),
    ],
)
def _domain_mask_sc(w_hbm, p_hbm, out_hbm, w_v, b_v, s_w, s_b, s_o):
    wid = lax.axis_index("s") * _NC + lax.axis_index("c")
    col0 = pl.multiple_of(wid * _PANEL, _PANEL)

    zero = jnp.zeros((_L,), jnp.float32)

    in_w, in_b, out_h = [], [], []
    for c in range(_NCHUNK):
        col = col0 + c * _CHUNK
        in_w.append(pltpu.async_copy(
            w_hbm.at[pl.ds(0, _SC_R), pl.ds(col, _CHUNK)],
            w_v.at[c], s_w.at[c]))
        in_b.append(pltpu.async_copy(
            p_hbm.at[pl.ds(col, _CHUNK)], b_v.at[c], s_b.at[c]))

    for c in range(_NCHUNK):
        in_w[c].wait()
        in_b[c].wait()

        @plsc.parallel_loop(0, _CHUNK // _L)
        def _body(j):
            words = b_v[c, pl.ds(j * _L, _L)]
            for r in range(_SC_R):
                vec = w_v[c, r, pl.ds(j * _L, _L)]
                hit = (words << (31 - r)) < 0
                w_v[c, r, pl.ds(j * _L, _L)] = jnp.where(hit, vec, zero)

        out_h.append(pltpu.async_copy(
            w_v.at[c],
            out_hbm.at[pl.ds(0, _SC_R), pl.ds(col0 + c * _CHUNK, _CHUNK)],
            s_o.at[c]))

    for h in out_h:
        h.wait()


def _tc_body(w_ref, m_ref, o_ref):
    o_ref[...] = jnp.where(m_ref[...] != 0, w_ref[...], jnp.float32(0.0))


_TC_GRID = (_R - _SC_R) // 16

_tc_rows = pl.pallas_call(
    _tc_body,
    grid=(_TC_GRID,),
    in_specs=[
        pl.BlockSpec((16, _C), lambda i: (i + _SC_R // 16, 0)),
        pl.BlockSpec((16, _C), lambda i: (i + _SC_R // 16, 0)),
    ],
    out_specs=pl.BlockSpec((16, _C), lambda i: (i + _SC_R // 16, 0)),
    out_shape=jax.ShapeDtypeStruct((_R, _C), jnp.float32),
)


def _pack_rows(m8):
    wt = jnp.left_shift(
        jnp.int32(1), jnp.arange(_SC_R, dtype=jnp.int32)
    )[:, None]
    return jnp.sum(m8[:_SC_R].astype(jnp.int32) * wt, axis=0)


def kernel(w, mask):
    m8 = mask.view(jnp.int8)
    sc_out = _domain_mask_sc(w, _pack_rows(m8))
    tc_out = _tc_rows(w, m8)
    return lax.dynamic_update_slice(tc_out, sc_out, (0, 0))
